# baseline (speedup 1.0000x reference)
import functools

import jax
import jax.numpy as jnp
from jax import lax
from jax.experimental import pallas as pl
from jax.experimental.pallas import tpu as pltpu
from jax.experimental.pallas import tpu_sc as plsc

N = 10000
E = 320000
D = 128

NC = 2
NS = 16
NW = NC * NS

CHUNK = 128
CPW0 = 102
CPW1 = 57
NCHUNKS = NS * (CPW0 + CPW1)
EPAD = NCHUNKS * CHUNK

ROWS_MAIN = 632
ROWS_LAST = N - (NS - 1) * ROWS_MAIN

_GDN = lax.GatherDimensionNumbers(
    offset_dims=(), collapsed_slice_dims=(0,), start_index_map=(0,))


def _sc_aggregate_body(sup_hbm, src_hbm, dst_hbm, adj_hbm, zz_hbm, out_hbm,
                       acc, rows, ia, da, aa, isem, gsem, ssem):
    c = lax.axis_index("c")
    s = lax.axis_index("s")
    kbase = jnp.where(c == 0, s * CPW0, NS * CPW0 + s * CPW1)
    cpw = jnp.where(c == 0, CPW0, CPW1)
    ni = jnp.where(c == 0, CPW0 // 3, CPW1 // 3)

    def idxcopy(k, p):
        off = (kbase + k) * CHUNK
        pltpu.async_copy(src_hbm.at[pl.ds(off, CHUNK)], ia[p], isem[p])
        pltpu.async_copy(dst_hbm.at[pl.ds(off, CHUNK)], da[p], isem[p])
        pltpu.async_copy(adj_hbm.at[pl.ds(off, CHUNK)], aa[p], isem[p])

    def wait_idxcopy(p):
        pltpu.make_async_copy(src_hbm.at[pl.ds(0, CHUNK)], ia[p], isem[p]).wait()
        pltpu.make_async_copy(dst_hbm.at[pl.ds(0, CHUNK)], da[p], isem[p]).wait()
        pltpu.make_async_copy(adj_hbm.at[pl.ds(0, CHUNK)], aa[p], isem[p]).wait()

    def gather(p):
        pltpu.async_copy(sup_hbm.at[ia[p]], rows[p], gsem[p])

    def wait_gather(p):
        pltpu.make_async_copy(sup_hbm.at[ia[p]], rows[p], gsem[p]).wait()

    def scatter(p):
        pltpu.async_copy(rows[p], acc.at[da[p]], ssem[p], add=True)

    def wait_scatter(p):
        pltpu.make_async_copy(rows[p], acc.at[da[p]], ssem[p]).wait()

    def scale(p):
        buf = rows[p]
        adjr = aa[p]

        def scale_group(g, carry):
            av = adjr[pl.ds(g * 16, 16)]
            for j in range(16):
                e = g * 16 + j
                a = lax.gather(av, jnp.full((16, 1), j, jnp.int32), _GDN,
                               slice_sizes=(1,),
                               mode=lax.GatherScatterMode.PROMISE_IN_BOUNDS)
                for col in range(D // 16):
                    buf[e, pl.ds(col * 16, 16)] = (
                        buf[e, pl.ds(col * 16, 16)] * a)
            return carry

        lax.fori_loop(0, CHUNK // 16, scale_group, 0)

    idxcopy(0, 0)
    idxcopy(1, 1)

    @pl.when(s < NS - 1)
    def _():
        pltpu.sync_copy(zz_hbm, acc.at[pl.ds(s * ROWS_MAIN, ROWS_MAIN)])

    @pl.when(s == NS - 1)
    def _():
        pltpu.sync_copy(zz_hbm.at[pl.ds(0, ROWS_LAST)],
                        acc.at[pl.ds((NS - 1) * ROWS_MAIN, ROWS_LAST)])

    wait_idxcopy(0)
    gather(0)
    plsc.subcore_barrier()

    def body(i, carry):
        for p in range(3):
            k = 3 * i + p
            wait_gather(p)
            scale(p)
            scatter(p)
            r2 = (p + 2) % 3
            r1 = (p + 1) % 3

            @pl.when(k > 0)
            def _():
                wait_scatter(r2)

            @pl.when(k + 2 < cpw)
            def _():
                idxcopy(k + 2, r2)

            @pl.when(k + 1 < cpw)
            def _():
                wait_idxcopy(r1)
                gather(r1)
        return carry

    lax.fori_loop(0, ni, body, 0)
    wait_scatter(2)

    plsc.subcore_barrier()

    @pl.when(s < NS - 1)
    def _():
        pltpu.sync_copy(acc.at[pl.ds(s * ROWS_MAIN, ROWS_MAIN)],
                        out_hbm.at[c, pl.ds(s * ROWS_MAIN, ROWS_MAIN)])

    @pl.when(s == NS - 1)
    def _():
        pltpu.sync_copy(acc.at[pl.ds((NS - 1) * ROWS_MAIN, ROWS_LAST)],
                        out_hbm.at[c, pl.ds((NS - 1) * ROWS_MAIN, ROWS_LAST)])


_sc_aggregate = functools.partial(
    pl.kernel,
    out_type=jax.ShapeDtypeStruct((NC, N, D), jnp.float32),
    mesh=plsc.VectorSubcoreMesh(core_axis_name="c", subcore_axis_name="s"),
    scratch_types=[
        pltpu.VMEM_SHARED((N, D), jnp.float32),
        [pltpu.VMEM((CHUNK, D), jnp.float32) for _ in range(3)],
        [pltpu.VMEM((CHUNK,), jnp.int32) for _ in range(3)],
        [pltpu.VMEM((CHUNK,), jnp.int32) for _ in range(3)],
        [pltpu.VMEM((CHUNK,), jnp.float32) for _ in range(3)],
        [pltpu.SemaphoreType.DMA for _ in range(3)],
        [pltpu.SemaphoreType.DMA for _ in range(3)],
        [pltpu.SemaphoreType.DMA for _ in range(3)],
    ],
)(_sc_aggregate_body)


def _matmul_body(x_ref, wt_ref, b_ref, o_ref):
    o_ref[...] = jnp.dot(x_ref[...], wt_ref[...],
                         preferred_element_type=jnp.float32) + b_ref[...]


def _combine_body(p_ref, o_ref):
    o_ref[...] = jnp.tanh(p_ref[0] + p_ref[1])


_MM_BLOCK = 1000


def _support(x, wt, b2):
    return pl.pallas_call(
        _matmul_body,
        grid=(N // _MM_BLOCK,),
        in_specs=[
            pl.BlockSpec((_MM_BLOCK, D), lambda i: (i, 0)),
            pl.BlockSpec((D, D), lambda i: (0, 0)),
            pl.BlockSpec((1, D), lambda i: (0, 0)),
        ],
        out_specs=pl.BlockSpec((_MM_BLOCK, D), lambda i: (i, 0)),
        out_shape=jax.ShapeDtypeStruct((N, D), jnp.float32),
    )(x, wt, b2)


def _combine(partial):
    return pl.pallas_call(
        _combine_body,
        grid=(N // _MM_BLOCK,),
        in_specs=[pl.BlockSpec((NC, _MM_BLOCK, D), lambda i: (0, i, 0))],
        out_specs=pl.BlockSpec((_MM_BLOCK, D), lambda i: (i, 0)),
        out_shape=jax.ShapeDtypeStruct((N, D), jnp.float32),
    )(partial)


@jax.jit
def kernel(x, edge_index, adj_values, W, b):
    ei = edge_index.astype(jnp.int32)
    pad = EPAD - E
    src = jnp.concatenate([ei[1], jnp.zeros((pad,), jnp.int32)])
    dst = jnp.concatenate([ei[0], jnp.zeros((pad,), jnp.int32)])
    adj = jnp.concatenate([adj_values, jnp.zeros((pad,), jnp.float32)])
    support = _support(x, W.T, b.reshape(1, D))
    zz = jnp.zeros((ROWS_MAIN, D), jnp.float32)
    partial = _sc_aggregate(support, src, dst, adj, zz)
    return _combine(partial)


# speedup vs baseline: 1.4535x; 1.4535x over previous
import functools

import jax
import jax.numpy as jnp
from jax import lax
from jax.experimental import pallas as pl
from jax.experimental.pallas import tpu as pltpu
from jax.experimental.pallas import tpu_sc as plsc

N = 10000
E = 320000
D = 128

NC = 2
NS = 16
NW = NC * NS

CHUNK = 112
NCHUNKS = 2880
EPAD = NCHUNKS * CHUNK
CPW0 = 114
CPW1 = 66

ROWS_MAIN = 632
ROWS_LAST = N - (NS - 1) * ROWS_MAIN

_GDN = lax.GatherDimensionNumbers(
    offset_dims=(), collapsed_slice_dims=(0,), start_index_map=(0,))


def _sc_aggregate_body(sup_hbm, src_hbm, dst_hbm, adj_hbm, zz_hbm, out_hbm,
                       acc, rows, ia, da, aa, isem, gsem, ssem):
    c = lax.axis_index("c")
    s = lax.axis_index("s")
    kbase = jnp.where(c == 0, s * CPW0, NS * CPW0 + s * CPW1)
    cpw = jnp.where(c == 0, CPW0, CPW1)
    ni = jnp.where(c == 0, CPW0 // 3, CPW1 // 3)

    def idxcopy(k, p):
        off = (kbase + k) * CHUNK
        pltpu.async_copy(src_hbm.at[pl.ds(off, CHUNK)], ia[p], isem[p])
        pltpu.async_copy(dst_hbm.at[pl.ds(off, CHUNK)], da[p], isem[p])
        pltpu.async_copy(adj_hbm.at[pl.ds(off, CHUNK)], aa[p], isem[p])

    def wait_idxcopy(p):
        pltpu.make_async_copy(src_hbm.at[pl.ds(0, CHUNK)], ia[p], isem[p]).wait()
        pltpu.make_async_copy(dst_hbm.at[pl.ds(0, CHUNK)], da[p], isem[p]).wait()
        pltpu.make_async_copy(adj_hbm.at[pl.ds(0, CHUNK)], aa[p], isem[p]).wait()

    def gather(p):
        pltpu.async_copy(sup_hbm.at[ia[p]], rows[p], gsem[p])

    def wait_gather(p):
        pltpu.make_async_copy(sup_hbm.at[ia[p]], rows[p], gsem[p]).wait()

    def scatter(p):
        pltpu.async_copy(rows[p], acc.at[da[p]], ssem[p], add=True)

    def wait_scatter(p):
        pltpu.make_async_copy(rows[p], acc.at[da[p]], ssem[p]).wait()

    def scale(p):
        buf = rows[p]
        adjr = aa[p]

        def scale_group(g, carry):
            av = adjr[pl.ds(g * 16, 16)]
            for j in range(16):
                e = g * 16 + j
                a = lax.gather(av, jnp.full((16, 1), j, jnp.int32), _GDN,
                               slice_sizes=(1,),
                               mode=lax.GatherScatterMode.PROMISE_IN_BOUNDS)
                for col in range(D // 16):
                    buf[e, pl.ds(col * 16, 16)] = (
                        buf[e, pl.ds(col * 16, 16)] * a)
            return carry

        lax.fori_loop(0, CHUNK // 16, scale_group, 0)

    idxcopy(0, 0)
    idxcopy(1, 1)

    @pl.when(s < NS - 1)
    def _():
        pltpu.sync_copy(zz_hbm, acc.at[pl.ds(s * ROWS_MAIN, ROWS_MAIN)])

    @pl.when(s == NS - 1)
    def _():
        pltpu.sync_copy(zz_hbm.at[pl.ds(0, ROWS_LAST)],
                        acc.at[pl.ds((NS - 1) * ROWS_MAIN, ROWS_LAST)])

    wait_idxcopy(0)
    gather(0)
    plsc.subcore_barrier()

    def body(i, carry):
        for p in range(3):
            k = 3 * i + p
            wait_gather(p)
            scale(p)
            scatter(p)
            r2 = (p + 2) % 3
            r1 = (p + 1) % 3

            @pl.when(k > 0)
            def _():
                wait_scatter(r2)

            @pl.when(k + 2 < cpw)
            def _():
                idxcopy(k + 2, r2)

            @pl.when(k + 1 < cpw)
            def _():
                wait_idxcopy(r1)
                gather(r1)
        return carry

    lax.fori_loop(0, ni, body, 0)
    wait_scatter(2)

    plsc.subcore_barrier()

    @pl.when(s < NS - 1)
    def _():
        pltpu.sync_copy(acc.at[pl.ds(s * ROWS_MAIN, ROWS_MAIN)],
                        out_hbm.at[c, pl.ds(s * ROWS_MAIN, ROWS_MAIN)])

    @pl.when(s == NS - 1)
    def _():
        pltpu.sync_copy(acc.at[pl.ds((NS - 1) * ROWS_MAIN, ROWS_LAST)],
                        out_hbm.at[c, pl.ds((NS - 1) * ROWS_MAIN, ROWS_LAST)])


_sc_aggregate = functools.partial(
    pl.kernel,
    out_type=jax.ShapeDtypeStruct((NC, N, D), jnp.float32),
    mesh=plsc.VectorSubcoreMesh(core_axis_name="c", subcore_axis_name="s"),
    scratch_types=[
        pltpu.VMEM_SHARED((N, D), jnp.float32),
        [pltpu.VMEM((CHUNK, D), jnp.float32) for _ in range(3)],
        [pltpu.VMEM((CHUNK,), jnp.int32) for _ in range(3)],
        [pltpu.VMEM((CHUNK,), jnp.int32) for _ in range(3)],
        [pltpu.VMEM((CHUNK,), jnp.float32) for _ in range(3)],
        [pltpu.SemaphoreType.DMA for _ in range(3)],
        [pltpu.SemaphoreType.DMA for _ in range(3)],
        [pltpu.SemaphoreType.DMA for _ in range(3)],
    ],
)(_sc_aggregate_body)


def _matmul_body(x_ref, wt_ref, b_ref, o_ref):
    o_ref[...] = jnp.dot(x_ref[...], wt_ref[...],
                         preferred_element_type=jnp.float32) + b_ref[...]


def _combine_body(p_ref, o_ref):
    o_ref[...] = jnp.tanh(p_ref[0] + p_ref[1])


_MM_BLOCK = 1000


def _support(x, wt, b2):
    return pl.pallas_call(
        _matmul_body,
        grid=(N // _MM_BLOCK,),
        in_specs=[
            pl.BlockSpec((_MM_BLOCK, D), lambda i: (i, 0)),
            pl.BlockSpec((D, D), lambda i: (0, 0)),
            pl.BlockSpec((1, D), lambda i: (0, 0)),
        ],
        out_specs=pl.BlockSpec((_MM_BLOCK, D), lambda i: (i, 0)),
        out_shape=jax.ShapeDtypeStruct((N, D), jnp.float32),
    )(x, wt, b2)


def _combine(partial):
    return pl.pallas_call(
        _combine_body,
        grid=(N // _MM_BLOCK,),
        in_specs=[pl.BlockSpec((NC, _MM_BLOCK, D), lambda i: (0, i, 0))],
        out_specs=pl.BlockSpec((_MM_BLOCK, D), lambda i: (i, 0)),
        out_shape=jax.ShapeDtypeStruct((N, D), jnp.float32),
    )(partial)


@jax.jit
def kernel(x, edge_index, adj_values, W, b):
    ei = edge_index.astype(jnp.int32)
    pad = EPAD - E
    src = jnp.concatenate([ei[1], jnp.zeros((pad,), jnp.int32)])
    dst = jnp.concatenate([ei[0], jnp.zeros((pad,), jnp.int32)])
    adj = jnp.concatenate([adj_values, jnp.zeros((pad,), jnp.float32)])
    support = _support(x, W.T, b.reshape(1, D))
    zz = jnp.zeros((ROWS_MAIN, D), jnp.float32)
    partial = _sc_aggregate(support, src, dst, adj, zz)
    return _combine(partial)
